# baseline (speedup 1.0000x reference)
import functools

import jax
import jax.numpy as jnp
from jax import lax
from jax.experimental import pallas as pl
from jax.experimental.pallas import tpu as pltpu
from jax.experimental.pallas import tpu_sc as plsc

NC = 2
NS = 16
NW = NC * NS

B = 4096
L = 200
D = 64
V = 1000000
CH = (128, 72)
OFF = (0, 128)
DP = 128
LP = 256
N2 = 500224
TBLK = 512
RPW = B // NW
NBUF = 4


def _tr_body(xa_ref, xb_ref, o_ref):
    o_ref[:, 0:D] = xa_ref[...].T
    o_ref[:, D:DP] = xb_ref[...].T


_table = pl.pallas_call(
    _tr_body,
    grid=(N2 // TBLK,),
    in_specs=[
        pl.BlockSpec((D, TBLK), lambda i: (0, i)),
        pl.BlockSpec((D, TBLK), lambda i: (0, i + N2 // TBLK)),
    ],
    out_specs=pl.BlockSpec((TBLK, DP), lambda i: (i, 0)),
    out_shape=jax.ShapeDtypeStruct((N2, DP), jnp.float32),
)


def _pool_body(text_ref, tab_ref, out_ref, idx_v, bufs, out_v, s0, s1, s2, s3):
    sems = (s0, s1, s2, s3)
    wid = lax.axis_index("s") * NC + lax.axis_index("c")

    pltpu.sync_copy(text_ref.at[pl.ds(wid * RPW, RPW)], idx_v)

    def prep_r(r, carry):
        def prep_g(g, carry2):
            sl = pl.ds(g * 16, 16)
            t = idx_v[r, sl]
            idx_v[r, sl] = jnp.where(t >= N2, 2 * t - (2 * N2 - 1), 2 * t)
            return carry2
        return lax.fori_loop(0, LP // 16, prep_g, carry)

    lax.fori_loop(0, RPW, prep_r, 0)

    def gather(r, h, b):
        dst = bufs.at[b] if CH[h] == CH[0] else bufs.at[b].at[pl.ds(0, CH[h])]
        return pltpu.make_async_copy(
            tab_ref.at[idx_v.at[r, pl.ds(OFF[h], CH[h])]], dst, sems[b])

    for b in range(NBUF):
        gather(b // 2, b % 2, b).start()

    def reduce_chunk(b, h):
        buf = bufs.at[b]

        def body(jj, carry):
            a0, a1, a2, a3 = carry
            for u in range(4):
                j = jj * 4 + u
                a0 = a0 + buf[j, pl.ds(0, 16)]
                a1 = a1 + buf[j, pl.ds(16, 16)]
                a2 = a2 + buf[j, pl.ds(32, 16)]
                a3 = a3 + buf[j, pl.ds(48, 16)]
            return a0, a1, a2, a3

        z = jnp.zeros((16,), jnp.float32)
        return lax.fori_loop(0, CH[h] // 4, body, (z, z, z, z))

    def outer(k, carry):
        for b in range(NBUF):
            r = k * (NBUF // 2) + (b // 2)
            h = b % 2
            gather(r, h, b).wait()
            a = reduce_chunk(b, h)
            if h == 0:
                for t in range(4):
                    out_v[r, pl.ds(16 * t, 16)] = a[t]
            else:
                for t in range(4):
                    out_v[r, pl.ds(16 * t, 16)] = (
                        out_v[r, pl.ds(16 * t, 16)] + a[t])

            @pl.when(k < (2 * RPW) // NBUF - 1)
            def _():
                gather(r + (NBUF // 2), h, b).start()

        return carry

    lax.fori_loop(0, (2 * RPW) // NBUF, outer, 0)
    pltpu.sync_copy(out_v, out_ref.at[pl.ds(wid * RPW, RPW)])


_pool = functools.partial(
    pl.kernel,
    out_type=jax.ShapeDtypeStruct((B, D), jnp.float32),
    mesh=plsc.VectorSubcoreMesh(
        core_axis_name="c", subcore_axis_name="s",
        num_cores=NC, num_subcores=NS),
    scratch_types=[
        pltpu.VMEM((RPW, LP), jnp.int32),
        pltpu.VMEM((NBUF, CH[0], D), jnp.float32),
        pltpu.VMEM((RPW, D), jnp.float32),
        pltpu.SemaphoreType.DMA,
        pltpu.SemaphoreType.DMA,
        pltpu.SemaphoreType.DMA,
        pltpu.SemaphoreType.DMA,
    ],
    compiler_params=pltpu.CompilerParams(use_tc_tiling_on_sc=False),
)(_pool_body)


def _mlp_body(x_ref, w1_ref, b1_ref, w2_ref, b2_ref, o_ref):
    x = x_ref[...] * (1.0 / L)
    h = lax.dot_general(x, w1_ref[...], (((1,), (1,)), ((), ())),
                        preferred_element_type=jnp.float32)
    h = jnp.maximum(h + b1_ref[...], 0.0)
    o = lax.dot_general(h, w2_ref[...], (((1,), (1,)), ((), ())),
                        preferred_element_type=jnp.float32)
    o_ref[...] = o + b2_ref[...]


def _mlp(x, w1, b1, w2, b2):
    return pl.pallas_call(
        _mlp_body,
        out_shape=jax.ShapeDtypeStruct((B, 10), jnp.float32),
    )(x, w1, b1.reshape(1, -1), w2, b2.reshape(1, -1))


def kernel(text, emb, W1, b1, W2, b2):
    text_p = jnp.pad(text.astype(jnp.int32), ((0, 0), (0, LP - L)))
    emb_t = emb.T
    table = _table(emb_t, emb_t).reshape(2 * N2, D)
    pooled_sum = _pool(text_p, table)
    return _mlp(pooled_sum, W1, b1, W2, b2)


# speedup vs baseline: 1.8135x; 1.8135x over previous
import functools

import jax
import jax.numpy as jnp
from jax import lax
from jax.experimental import pallas as pl
from jax.experimental.pallas import tpu as pltpu
from jax.experimental.pallas import tpu_sc as plsc

NC = 2
NS = 16
NW = NC * NS

B = 4096
L = 200
D = 64
V = 1000000
CH = (128, 72)
OFF = (0, 128)
DP = 128
LP = 256
N2 = 501760
TBLK = 2048
RPW = B // NW
NBUF = 4


def _tr_body(xa_ref, xb_ref, o_ref):
    o_ref[:, 0:D] = xa_ref[...].T
    o_ref[:, D:DP] = xb_ref[...].T


_table = pl.pallas_call(
    _tr_body,
    grid=(N2 // TBLK,),
    in_specs=[
        pl.BlockSpec((D, TBLK), lambda i: (0, i)),
        pl.BlockSpec(
            (D, TBLK),
            lambda i: (0, jnp.minimum(i + N2 // TBLK, V // TBLK))),
    ],
    out_specs=pl.BlockSpec((TBLK, DP), lambda i: (i, 0)),
    out_shape=jax.ShapeDtypeStruct((N2, DP), jnp.float32),
)


def _pool_body(text_ref, tab_ref, out_ref, idx_v, bufs, out_v, s0, s1, s2, s3):
    sems = (s0, s1, s2, s3)
    wid = lax.axis_index("s") * NC + lax.axis_index("c")

    pltpu.sync_copy(text_ref.at[pl.ds(wid * RPW, RPW)], idx_v)

    def prep_r(r, carry):
        def prep_g(g, carry2):
            sl = pl.ds(g * 16, 16)
            t = idx_v[r, sl]
            idx_v[r, sl] = jnp.where(t >= N2, 2 * t - (2 * N2 - 1), 2 * t)
            return carry2
        return lax.fori_loop(0, LP // 16, prep_g, carry)

    lax.fori_loop(0, RPW, prep_r, 0)

    def gather(r, h, b):
        dst = bufs.at[b] if CH[h] == CH[0] else bufs.at[b].at[pl.ds(0, CH[h])]
        return pltpu.make_async_copy(
            tab_ref.at[idx_v.at[r, pl.ds(OFF[h], CH[h])]], dst, sems[b])

    for b in range(NBUF):
        gather(b // 2, b % 2, b).start()

    def reduce_chunk(b, h):
        buf = bufs.at[b]

        def body(jj, carry):
            a0, a1, a2, a3 = carry
            for u in range(4):
                j = jj * 4 + u
                a0 = a0 + buf[j, pl.ds(0, 16)]
                a1 = a1 + buf[j, pl.ds(16, 16)]
                a2 = a2 + buf[j, pl.ds(32, 16)]
                a3 = a3 + buf[j, pl.ds(48, 16)]
            return a0, a1, a2, a3

        z = jnp.zeros((16,), jnp.float32)
        return lax.fori_loop(0, CH[h] // 4, body, (z, z, z, z))

    def outer(k, carry):
        for b in range(NBUF):
            r = k * (NBUF // 2) + (b // 2)
            h = b % 2
            gather(r, h, b).wait()
            a = reduce_chunk(b, h)
            if h == 0:
                for t in range(4):
                    out_v[r, pl.ds(16 * t, 16)] = a[t]
            else:
                for t in range(4):
                    out_v[r, pl.ds(16 * t, 16)] = (
                        out_v[r, pl.ds(16 * t, 16)] + a[t])

            @pl.when(k < (2 * RPW) // NBUF - 1)
            def _():
                gather(r + (NBUF // 2), h, b).start()

        return carry

    lax.fori_loop(0, (2 * RPW) // NBUF, outer, 0)
    pltpu.sync_copy(out_v, out_ref.at[pl.ds(wid * RPW, RPW)])


_pool = functools.partial(
    pl.kernel,
    out_type=jax.ShapeDtypeStruct((B, D), jnp.float32),
    mesh=plsc.VectorSubcoreMesh(
        core_axis_name="c", subcore_axis_name="s",
        num_cores=NC, num_subcores=NS),
    scratch_types=[
        pltpu.VMEM((RPW, LP), jnp.int32),
        pltpu.VMEM((NBUF, CH[0], D), jnp.float32),
        pltpu.VMEM((RPW, D), jnp.float32),
        pltpu.SemaphoreType.DMA,
        pltpu.SemaphoreType.DMA,
        pltpu.SemaphoreType.DMA,
        pltpu.SemaphoreType.DMA,
    ],
    compiler_params=pltpu.CompilerParams(use_tc_tiling_on_sc=False),
)(_pool_body)


def _mlp_body(x_ref, w1_ref, b1_ref, w2_ref, b2_ref, o_ref):
    x = x_ref[...] * (1.0 / L)
    h = lax.dot_general(x, w1_ref[...], (((1,), (1,)), ((), ())),
                        preferred_element_type=jnp.float32)
    h = jnp.maximum(h + b1_ref[...], 0.0)
    o = lax.dot_general(h, w2_ref[...], (((1,), (1,)), ((), ())),
                        preferred_element_type=jnp.float32)
    o_ref[...] = o + b2_ref[...]


def _mlp(x, w1, b1, w2, b2):
    return pl.pallas_call(
        _mlp_body,
        out_shape=jax.ShapeDtypeStruct((B, 10), jnp.float32),
    )(x, w1, b1.reshape(1, -1), w2, b2.reshape(1, -1))


def kernel(text, emb, W1, b1, W2, b2):
    text_p = jnp.pad(text.astype(jnp.int32), ((0, 0), (0, LP - L)))
    emb_t = emb.T
    table = _table(emb_t, emb_t).reshape(2 * N2, D)
    pooled_sum = _pool(text_p, table)
    return _mlp(pooled_sum, W1, b1, W2, b2)


# speedup vs baseline: 2.2571x; 1.2446x over previous
import functools

import jax
import jax.numpy as jnp
from jax import lax
from jax.experimental import pallas as pl
from jax.experimental.pallas import tpu as pltpu
from jax.experimental.pallas import tpu_sc as plsc

NC = 2
NS = 16
NW = NC * NS

B = 4096
L = 200
D = 64
V = 1000000
CH = (128, 72)
OFF = (0, 128)
DP = 128
LP = 256
N2 = 503808
TBLK = 4096
RPW = B // NW
NBUF = 8


def _tr_body(xa_ref, xb_ref, o_ref):
    o_ref[:, 0:D] = xa_ref[...].T
    o_ref[:, D:DP] = xb_ref[...].T


_table = pl.pallas_call(
    _tr_body,
    grid=(N2 // TBLK,),
    in_specs=[
        pl.BlockSpec((D, TBLK), lambda i: (0, i)),
        pl.BlockSpec(
            (D, TBLK),
            lambda i: (0, jnp.minimum(i + N2 // TBLK, V // TBLK))),
    ],
    out_specs=pl.BlockSpec((TBLK, DP), lambda i: (i, 0)),
    out_shape=jax.ShapeDtypeStruct((N2, DP), jnp.float32),
)


def _pool_body(text_ref, tab_ref, out_ref, idx_v, bufs, out_v,
               s0, s1, s2, s3, s4, s5, s6, s7):
    sems = (s0, s1, s2, s3, s4, s5, s6, s7)
    wid = lax.axis_index("s") * NC + lax.axis_index("c")

    pltpu.sync_copy(text_ref.at[pl.ds(wid * RPW, RPW)], idx_v)

    def prep_r(r, carry):
        def prep_g(g, carry2):
            sl = pl.ds(g * 16, 16)
            t = idx_v[r, sl]
            idx_v[r, sl] = jnp.where(t >= N2, 2 * t - (2 * N2 - 1), 2 * t)
            return carry2
        return lax.fori_loop(0, LP // 16, prep_g, carry)

    lax.fori_loop(0, RPW, prep_r, 0)

    def gather(r, h, b):
        dst = bufs.at[b] if CH[h] == CH[0] else bufs.at[b].at[pl.ds(0, CH[h])]
        return pltpu.make_async_copy(
            tab_ref.at[idx_v.at[r, pl.ds(OFF[h], CH[h])]], dst, sems[b])

    for b in range(NBUF):
        gather(b // 2, b % 2, b).start()

    def reduce_chunk(b, h):
        buf = bufs.at[b]

        def body(jj, carry):
            a0, a1, a2, a3 = carry
            for u in range(8):
                j = jj * 8 + u
                a0 = a0 + buf[j, pl.ds(0, 16)]
                a1 = a1 + buf[j, pl.ds(16, 16)]
                a2 = a2 + buf[j, pl.ds(32, 16)]
                a3 = a3 + buf[j, pl.ds(48, 16)]
            return a0, a1, a2, a3

        z = jnp.zeros((16,), jnp.float32)
        return lax.fori_loop(0, CH[h] // 8, body, (z, z, z, z))

    def outer(k, carry):
        for b in range(NBUF):
            r = k * (NBUF // 2) + (b // 2)
            h = b % 2
            gather(r, h, b).wait()
            a = reduce_chunk(b, h)
            if h == 0:
                for t in range(4):
                    out_v[r, pl.ds(16 * t, 16)] = a[t]
            else:
                for t in range(4):
                    out_v[r, pl.ds(16 * t, 16)] = (
                        out_v[r, pl.ds(16 * t, 16)] + a[t])

            @pl.when(k < (2 * RPW) // NBUF - 1)
            def _():
                gather(r + (NBUF // 2), h, b).start()

        return carry

    lax.fori_loop(0, (2 * RPW) // NBUF, outer, 0)
    pltpu.sync_copy(out_v, out_ref.at[pl.ds(wid * RPW, RPW)])


_pool = functools.partial(
    pl.kernel,
    out_type=jax.ShapeDtypeStruct((B, D), jnp.float32),
    mesh=plsc.VectorSubcoreMesh(
        core_axis_name="c", subcore_axis_name="s",
        num_cores=NC, num_subcores=NS),
    scratch_types=[
        pltpu.VMEM((RPW, LP), jnp.int32),
        pltpu.VMEM((NBUF, CH[0], D), jnp.float32),
        pltpu.VMEM((RPW, D), jnp.float32),
        pltpu.SemaphoreType.DMA,
        pltpu.SemaphoreType.DMA,
        pltpu.SemaphoreType.DMA,
        pltpu.SemaphoreType.DMA,
        pltpu.SemaphoreType.DMA,
        pltpu.SemaphoreType.DMA,
        pltpu.SemaphoreType.DMA,
        pltpu.SemaphoreType.DMA,
    ],
    compiler_params=pltpu.CompilerParams(use_tc_tiling_on_sc=False),
)(_pool_body)


def _mlp_body(x_ref, w1_ref, b1_ref, w2_ref, b2_ref, o_ref):
    x = x_ref[...] * (1.0 / L)
    h = lax.dot_general(x, w1_ref[...], (((1,), (1,)), ((), ())),
                        preferred_element_type=jnp.float32)
    h = jnp.maximum(h + b1_ref[...], 0.0)
    o = lax.dot_general(h, w2_ref[...], (((1,), (1,)), ((), ())),
                        preferred_element_type=jnp.float32)
    o_ref[...] = o + b2_ref[...]


def _mlp(x, w1, b1, w2, b2):
    return pl.pallas_call(
        _mlp_body,
        out_shape=jax.ShapeDtypeStruct((B, 10), jnp.float32),
    )(x, w1, b1.reshape(1, -1), w2, b2.reshape(1, -1))


def kernel(text, emb, W1, b1, W2, b2):
    text_p = jnp.pad(text.astype(jnp.int32), ((0, 0), (0, LP - L)))
    emb_t = emb.T
    table = _table(emb_t, emb_t).reshape(2 * N2, D)
    pooled_sum = _pool(text_p, table)
    return _mlp(pooled_sum, W1, b1, W2, b2)


# speedup vs baseline: 2.5028x; 1.1089x over previous
import functools

import jax
import jax.numpy as jnp
from jax import lax
from jax.experimental import pallas as pl
from jax.experimental.pallas import tpu as pltpu
from jax.experimental.pallas import tpu_sc as plsc

NC = 2
NS = 16
NW = NC * NS

B = 4096
L = 200
D = 64
V = 1000000
CH = (128, 72)
OFF = (0, 128)
DP = 128
LP = 256
N2 = 507904
TBLK = 8192
RPW = B // NW
NBUF = 8


def _tr_body(xa_ref, xb_ref, o_ref):
    o_ref[:, 0:D] = xa_ref[...].T
    o_ref[:, D:DP] = xb_ref[...].T


_table = pl.pallas_call(
    _tr_body,
    grid=(N2 // TBLK,),
    in_specs=[
        pl.BlockSpec((D, TBLK), lambda i: (0, i)),
        pl.BlockSpec(
            (D, TBLK),
            lambda i: (0, jnp.minimum(i + N2 // TBLK, V // TBLK))),
    ],
    out_specs=pl.BlockSpec((TBLK, DP), lambda i: (i, 0)),
    out_shape=jax.ShapeDtypeStruct((N2, DP), jnp.float32),
)


def _pool_body(text_ref, tab_ref, out_ref, idx_v, bufs, out_v,
               s0, s1, s2, s3, s4, s5, s6, s7):
    sems = (s0, s1, s2, s3, s4, s5, s6, s7)
    wid = lax.axis_index("s") * NC + lax.axis_index("c")

    pltpu.sync_copy(text_ref.at[pl.ds(wid * RPW, RPW)], idx_v)

    def prep_r(r, carry):
        def prep_g(g, carry2):
            sl = pl.ds(g * 16, 16)
            t = idx_v[r, sl]
            idx_v[r, sl] = jnp.where(t >= N2, 2 * t - (2 * N2 - 1), 2 * t)
            return carry2
        return lax.fori_loop(0, LP // 16, prep_g, carry)

    lax.fori_loop(0, RPW, prep_r, 0)

    def gather(r, h, b):
        dst = bufs.at[b] if CH[h] == CH[0] else bufs.at[b].at[pl.ds(0, CH[h])]
        return pltpu.make_async_copy(
            tab_ref.at[idx_v.at[r, pl.ds(OFF[h], CH[h])]], dst, sems[b])

    for b in range(NBUF):
        gather(b // 2, b % 2, b).start()

    def reduce_chunk(b, h):
        buf = bufs.at[b]

        def body(jj, carry):
            a0, a1, a2, a3 = carry
            for u in range(8):
                j = jj * 8 + u
                a0 = a0 + buf[j, pl.ds(0, 16)]
                a1 = a1 + buf[j, pl.ds(16, 16)]
                a2 = a2 + buf[j, pl.ds(32, 16)]
                a3 = a3 + buf[j, pl.ds(48, 16)]
            return a0, a1, a2, a3

        z = jnp.zeros((16,), jnp.float32)
        return lax.fori_loop(0, CH[h] // 8, body, (z, z, z, z))

    def outer(k, carry):
        for b in range(NBUF):
            r = k * (NBUF // 2) + (b // 2)
            h = b % 2
            gather(r, h, b).wait()
            a = reduce_chunk(b, h)
            if h == 0:
                for t in range(4):
                    out_v[r, pl.ds(16 * t, 16)] = a[t]
            else:
                for t in range(4):
                    out_v[r, pl.ds(16 * t, 16)] = (
                        out_v[r, pl.ds(16 * t, 16)] + a[t])

            @pl.when(k < (2 * RPW) // NBUF - 1)
            def _():
                gather(r + (NBUF // 2), h, b).start()

        return carry

    lax.fori_loop(0, (2 * RPW) // NBUF, outer, 0)
    pltpu.sync_copy(out_v, out_ref.at[pl.ds(wid * RPW, RPW)])


_pool = functools.partial(
    pl.kernel,
    out_type=jax.ShapeDtypeStruct((B, D), jnp.float32),
    mesh=plsc.VectorSubcoreMesh(
        core_axis_name="c", subcore_axis_name="s",
        num_cores=NC, num_subcores=NS),
    scratch_types=[
        pltpu.VMEM((RPW, LP), jnp.int32),
        pltpu.VMEM((NBUF, CH[0], D), jnp.float32),
        pltpu.VMEM((RPW, D), jnp.float32),
        pltpu.SemaphoreType.DMA,
        pltpu.SemaphoreType.DMA,
        pltpu.SemaphoreType.DMA,
        pltpu.SemaphoreType.DMA,
        pltpu.SemaphoreType.DMA,
        pltpu.SemaphoreType.DMA,
        pltpu.SemaphoreType.DMA,
        pltpu.SemaphoreType.DMA,
    ],
    compiler_params=pltpu.CompilerParams(use_tc_tiling_on_sc=False),
)(_pool_body)


def _mlp_body(x_ref, w1_ref, b1_ref, w2_ref, b2_ref, o_ref):
    x = x_ref[...] * (1.0 / L)
    h = lax.dot_general(x, w1_ref[...], (((1,), (1,)), ((), ())),
                        preferred_element_type=jnp.float32)
    h = jnp.maximum(h + b1_ref[...], 0.0)
    o = lax.dot_general(h, w2_ref[...], (((1,), (1,)), ((), ())),
                        preferred_element_type=jnp.float32)
    o_ref[...] = o + b2_ref[...]


def _mlp(x, w1, b1, w2, b2):
    return pl.pallas_call(
        _mlp_body,
        out_shape=jax.ShapeDtypeStruct((B, 10), jnp.float32),
    )(x, w1, b1.reshape(1, -1), w2, b2.reshape(1, -1))


def kernel(text, emb, W1, b1, W2, b2):
    text_p = jnp.pad(text.astype(jnp.int32), ((0, 0), (0, LP - L)))
    emb_t = emb.T
    table = _table(emb_t, emb_t).reshape(2 * N2, D)
    pooled_sum = _pool(text_p, table)
    return _mlp(pooled_sum, W1, b1, W2, b2)


# speedup vs baseline: 2.5884x; 1.0342x over previous
import functools

import jax
import jax.numpy as jnp
from jax import lax
from jax.experimental import pallas as pl
from jax.experimental.pallas import tpu as pltpu
from jax.experimental.pallas import tpu_sc as plsc

NC = 2
NS = 16
NW = NC * NS

B = 4096
L = 200
D = 64
V = 1000000
CH = (128, 72)
OFF = (0, 128)
DP = 128
LP = 256
N2 = 507904
TBLK = 16384
RPW = B // NW
NBUF = 8


def _tr_body(xa_ref, xb_ref, o_ref):
    o_ref[:, 0:D] = xa_ref[...].T
    o_ref[:, D:DP] = xb_ref[...].T


_table = pl.pallas_call(
    _tr_body,
    grid=(N2 // TBLK,),
    in_specs=[
        pl.BlockSpec((D, TBLK), lambda i: (0, i)),
        pl.BlockSpec(
            (D, TBLK),
            lambda i: (0, jnp.minimum(i + N2 // TBLK, V // TBLK))),
    ],
    out_specs=pl.BlockSpec((TBLK, DP), lambda i: (i, 0)),
    out_shape=jax.ShapeDtypeStruct((N2, DP), jnp.float32),
)


def _pool_body(text_ref, tab_ref, out_ref, idx_v, bufs, out_v,
               s0, s1, s2, s3, s4, s5, s6, s7):
    sems = (s0, s1, s2, s3, s4, s5, s6, s7)
    wid = lax.axis_index("s") * NC + lax.axis_index("c")

    pltpu.sync_copy(text_ref.at[pl.ds(wid * RPW, RPW)], idx_v)

    def prep_r(r, carry):
        def prep_g(g, carry2):
            sl = pl.ds(g * 16, 16)
            t = idx_v[r, sl]
            idx_v[r, sl] = jnp.where(t >= N2, 2 * t - (2 * N2 - 1), 2 * t)
            return carry2
        return lax.fori_loop(0, LP // 16, prep_g, carry)

    lax.fori_loop(0, RPW, prep_r, 0)

    def gather(r, h, b):
        dst = bufs.at[b] if CH[h] == CH[0] else bufs.at[b].at[pl.ds(0, CH[h])]
        return pltpu.make_async_copy(
            tab_ref.at[idx_v.at[r, pl.ds(OFF[h], CH[h])]], dst, sems[b])

    for b in range(NBUF):
        gather(b // 2, b % 2, b).start()

    def reduce_chunk(b, h):
        buf = bufs.at[b]

        def body(jj, carry):
            a0, a1, a2, a3 = carry
            for u in range(8):
                j = jj * 8 + u
                a0 = a0 + buf[j, pl.ds(0, 16)]
                a1 = a1 + buf[j, pl.ds(16, 16)]
                a2 = a2 + buf[j, pl.ds(32, 16)]
                a3 = a3 + buf[j, pl.ds(48, 16)]
            return a0, a1, a2, a3

        z = jnp.zeros((16,), jnp.float32)
        return lax.fori_loop(0, CH[h] // 8, body, (z, z, z, z))

    def outer(k, carry):
        for b in range(NBUF):
            r = k * (NBUF // 2) + (b // 2)
            h = b % 2
            gather(r, h, b).wait()
            a = reduce_chunk(b, h)
            if h == 0:
                for t in range(4):
                    out_v[r, pl.ds(16 * t, 16)] = a[t]
            else:
                for t in range(4):
                    out_v[r, pl.ds(16 * t, 16)] = (
                        out_v[r, pl.ds(16 * t, 16)] + a[t])

            @pl.when(k < (2 * RPW) // NBUF - 1)
            def _():
                gather(r + (NBUF // 2), h, b).start()

        return carry

    lax.fori_loop(0, (2 * RPW) // NBUF, outer, 0)
    pltpu.sync_copy(out_v, out_ref.at[pl.ds(wid * RPW, RPW)])


_pool = functools.partial(
    pl.kernel,
    out_type=jax.ShapeDtypeStruct((B, D), jnp.float32),
    mesh=plsc.VectorSubcoreMesh(
        core_axis_name="c", subcore_axis_name="s",
        num_cores=NC, num_subcores=NS),
    scratch_types=[
        pltpu.VMEM((RPW, LP), jnp.int32),
        pltpu.VMEM((NBUF, CH[0], D), jnp.float32),
        pltpu.VMEM((RPW, D), jnp.float32),
        pltpu.SemaphoreType.DMA,
        pltpu.SemaphoreType.DMA,
        pltpu.SemaphoreType.DMA,
        pltpu.SemaphoreType.DMA,
        pltpu.SemaphoreType.DMA,
        pltpu.SemaphoreType.DMA,
        pltpu.SemaphoreType.DMA,
        pltpu.SemaphoreType.DMA,
    ],
    compiler_params=pltpu.CompilerParams(use_tc_tiling_on_sc=False),
)(_pool_body)


def _mlp_body(x_ref, w1_ref, b1_ref, w2_ref, b2_ref, o_ref):
    x = x_ref[...] * (1.0 / L)
    h = lax.dot_general(x, w1_ref[...], (((1,), (1,)), ((), ())),
                        preferred_element_type=jnp.float32)
    h = jnp.maximum(h + b1_ref[...], 0.0)
    o = lax.dot_general(h, w2_ref[...], (((1,), (1,)), ((), ())),
                        preferred_element_type=jnp.float32)
    o_ref[...] = o + b2_ref[...]


def _mlp(x, w1, b1, w2, b2):
    return pl.pallas_call(
        _mlp_body,
        out_shape=jax.ShapeDtypeStruct((B, 10), jnp.float32),
    )(x, w1, b1.reshape(1, -1), w2, b2.reshape(1, -1))


def kernel(text, emb, W1, b1, W2, b2):
    text_p = jnp.pad(text.astype(jnp.int32), ((0, 0), (0, LP - L)))
    emb_t = emb.T
    table = _table(emb_t, emb_t).reshape(2 * N2, D)
    pooled_sum = _pool(text_p, table)
    return _mlp(pooled_sum, W1, b1, W2, b2)


# speedup vs baseline: 2.5923x; 1.0015x over previous
import functools

import jax
import jax.numpy as jnp
from jax import lax
from jax.experimental import pallas as pl
from jax.experimental.pallas import tpu as pltpu
from jax.experimental.pallas import tpu_sc as plsc

NC = 2
NS = 16
NW = NC * NS

B = 4096
L = 200
D = 64
V = 1000000
CH = (128, 72)
OFF = (0, 128)
DP = 128
LP = 256
N2 = 507904
TBLK = 16384
RPW = B // NW
NBUF = 8


def _tr_body(xa_ref, xb_ref, o_ref):
    o_ref[:, 0:D] = xa_ref[...].T
    o_ref[:, D:DP] = xb_ref[...].T


_table = pl.pallas_call(
    _tr_body,
    grid=(N2 // TBLK,),
    in_specs=[
        pl.BlockSpec((D, TBLK), lambda i: (0, i)),
        pl.BlockSpec(
            (D, TBLK),
            lambda i: (0, jnp.minimum(i + N2 // TBLK, V // TBLK))),
    ],
    out_specs=pl.BlockSpec((TBLK, DP), lambda i: (i, 0)),
    out_shape=jax.ShapeDtypeStruct((N2, DP), jnp.float32),
)


def _pool_body(text_ref, tab_ref, out_ref, idx_v, bufs, out_v,
               s0, s1, s2, s3, s4, s5, s6, s7):
    sems = (s0, s1, s2, s3, s4, s5, s6, s7)
    wid = lax.axis_index("s") * NC + lax.axis_index("c")

    pltpu.sync_copy(text_ref.at[pl.ds(wid * RPW, RPW)], idx_v)

    def prep_r(r, carry):
        for g in range(LP // 16):
            sl = pl.ds(g * 16, 16)
            t = idx_v[r, sl]
            idx_v[r, sl] = jnp.where(t >= N2, 2 * t - (2 * N2 - 1), 2 * t)
        return carry

    lax.fori_loop(0, RPW, prep_r, 0)

    def gather(r, h, b):
        dst = bufs.at[b] if CH[h] == CH[0] else bufs.at[b].at[pl.ds(0, CH[h])]
        return pltpu.make_async_copy(
            tab_ref.at[idx_v.at[r, pl.ds(OFF[h], CH[h])]], dst, sems[b])

    for b in range(NBUF):
        gather(b // 2, b % 2, b).start()

    def reduce_chunk(b, h):
        buf = bufs.at[b]

        def body(jj, carry):
            a0, a1, a2, a3 = carry
            for u in range(8):
                j = jj * 8 + u
                a0 = a0 + buf[j, pl.ds(0, 16)]
                a1 = a1 + buf[j, pl.ds(16, 16)]
                a2 = a2 + buf[j, pl.ds(32, 16)]
                a3 = a3 + buf[j, pl.ds(48, 16)]
            return a0, a1, a2, a3

        z = jnp.zeros((16,), jnp.float32)
        return lax.fori_loop(0, CH[h] // 8, body, (z, z, z, z))

    def outer(k, carry):
        for b in range(NBUF):
            r = k * (NBUF // 2) + (b // 2)
            h = b % 2
            gather(r, h, b).wait()
            a = reduce_chunk(b, h)
            if h == 0:
                for t in range(4):
                    out_v[r, pl.ds(16 * t, 16)] = a[t]
            else:
                for t in range(4):
                    out_v[r, pl.ds(16 * t, 16)] = (
                        out_v[r, pl.ds(16 * t, 16)] + a[t])

            @pl.when(k < (2 * RPW) // NBUF - 1)
            def _():
                gather(r + (NBUF // 2), h, b).start()

        return carry

    lax.fori_loop(0, (2 * RPW) // NBUF, outer, 0)
    pltpu.sync_copy(out_v, out_ref.at[pl.ds(wid * RPW, RPW)])


_pool = functools.partial(
    pl.kernel,
    out_type=jax.ShapeDtypeStruct((B, D), jnp.float32),
    mesh=plsc.VectorSubcoreMesh(
        core_axis_name="c", subcore_axis_name="s",
        num_cores=NC, num_subcores=NS),
    scratch_types=[
        pltpu.VMEM((RPW, LP), jnp.int32),
        pltpu.VMEM((NBUF, CH[0], D), jnp.float32),
        pltpu.VMEM((RPW, D), jnp.float32),
        pltpu.SemaphoreType.DMA,
        pltpu.SemaphoreType.DMA,
        pltpu.SemaphoreType.DMA,
        pltpu.SemaphoreType.DMA,
        pltpu.SemaphoreType.DMA,
        pltpu.SemaphoreType.DMA,
        pltpu.SemaphoreType.DMA,
        pltpu.SemaphoreType.DMA,
    ],
    compiler_params=pltpu.CompilerParams(use_tc_tiling_on_sc=False),
)(_pool_body)


def _mlp_body(x_ref, w1_ref, b1_ref, w2_ref, b2_ref, o_ref):
    x = x_ref[...] * (1.0 / L)
    h = lax.dot_general(x, w1_ref[...], (((1,), (1,)), ((), ())),
                        preferred_element_type=jnp.float32)
    h = jnp.maximum(h + b1_ref[...], 0.0)
    o = lax.dot_general(h, w2_ref[...], (((1,), (1,)), ((), ())),
                        preferred_element_type=jnp.float32)
    o_ref[...] = o + b2_ref[...]


def _mlp(x, w1, b1, w2, b2):
    return pl.pallas_call(
        _mlp_body,
        out_shape=jax.ShapeDtypeStruct((B, 10), jnp.float32),
    )(x, w1, b1.reshape(1, -1), w2, b2.reshape(1, -1))


def kernel(text, emb, W1, b1, W2, b2):
    text_p = jnp.pad(text.astype(jnp.int32), ((0, 0), (0, LP - L)))
    emb_t = emb.T
    table = _table(emb_t, emb_t).reshape(2 * N2, D)
    pooled_sum = _pool(text_p, table)
    return _mlp(pooled_sum, W1, b1, W2, b2)
